# baseline (speedup 1.0000x reference)
import functools

import jax
import jax.numpy as jnp
from jax import lax
from jax.experimental import pallas as pl
from jax.experimental.pallas import tpu as pltpu
from jax.experimental.pallas import tpu_sc as plsc

_F32 = jnp.float32
_NEG = -1e30


def _silu(v):
    return v * lax.logistic(v)


def _basis_cat(v):
    t = jnp.tanh(v)
    p0 = jnp.ones_like(t)
    p1 = t
    p2 = (3.0 * t * p1 - p0) * 0.5
    p3 = (5.0 * t * p2 - 2.0 * p1) * (1.0 / 3.0)
    return jnp.concatenate([p0, p1, p2, p3], axis=-1)


def _inorm_silu(y):
    m = jnp.mean(y, axis=0, keepdims=True)
    v = jnp.mean((y - m) * (y - m), axis=0, keepdims=True)
    return _silu((y - m) * lax.rsqrt(v + 1e-5))


def _stem_body(ph_ref, wg_ref, out_ref, a_ref, acc_ref):
    band = pl.program_id(1)
    ri = band * 16 + lax.broadcasted_iota(jnp.int32, (20, 115, 1), 0)
    ci = lax.broadcasted_iota(jnp.int32, (20, 115, 1), 1)
    for p in range(4):
        py, px = p // 2, p % 2
        xp = ph_ref[0, 0, :, :, p * 3:(p + 1) * 3]
        rr = 2 * ri + py
        cc = 2 * ci + px
        msk = ((rr >= 3) & (rr <= 226) & (cc >= 3) & (cc <= 226)
               ).astype(_F32)
        a_ref[:, :, p * 15:p * 15 + 3] = _silu(xp)
        a_ref[:, :, p * 15 + 3:p * 15 + 15] = _basis_cat(xp) * msk
    for g in range(16):
        dy, dx = g // 4, g % 4
        a = a_ref[dy:dy + 16, dx:dx + 112, :]
        a2 = a.reshape(1792, 60)
        c = jnp.dot(a2, wg_ref[g], preferred_element_type=_F32)
        if g == 0:
            acc_ref[...] = c
        else:
            acc_ref[...] += c
    y = acc_ref[:, :16] + acc_ref[:, 16:32]
    out_ref[0] = y


def _stem_call(ph_bands, wgm):
    return pl.pallas_call(
        _stem_body,
        grid=(8, 7),
        in_specs=[
            pl.BlockSpec((1, 1, 20, 115, 12), lambda b, k: (b, k, 0, 0, 0)),
            pl.BlockSpec((16, 60, 32), lambda b, k: (0, 0, 0)),
        ],
        out_specs=pl.BlockSpec((1, 1792, 16), lambda b, k: (b, k, 0)),
        out_shape=jax.ShapeDtypeStruct((8, 12544, 16), _F32),
        scratch_shapes=[
            pltpu.VMEM((20, 115, 60), _F32),
            pltpu.VMEM((1792, 32), _F32),
        ],
    )(ph_bands, wgm)


def _pool_body(t_ref, y_ref, out_ref):
    y = y_ref[0]
    m = jnp.mean(y, axis=0, keepdims=True)
    v = jnp.mean((y - m) * (y - m), axis=0, keepdims=True)
    rs = lax.rsqrt(v + 1e-5)
    tv = t_ref[0]
    mx = None
    for t in range(9):
        tap = tv[:, t * 16:(t + 1) * 16]
        s = jnp.where(tap < -1e29, _NEG, _silu((tap - m) * rs))
        mx = s if mx is None else jnp.maximum(mx, s)
    out_ref[0] = mx


def _pool_call(taps, y):
    b, n, c9 = taps.shape
    ny = y.shape[1]
    return pl.pallas_call(
        _pool_body,
        grid=(b,),
        in_specs=[
            pl.BlockSpec((1, n, c9), lambda i: (i, 0, 0)),
            pl.BlockSpec((1, ny, 16), lambda i: (i, 0, 0)),
        ],
        out_specs=pl.BlockSpec((1, n, 16), lambda i: (i, 0, 0)),
        out_shape=jax.ShapeDtypeStruct((b, n, 16), _F32),
    )(taps, y)


def _k1_body(x_ref, wb_ref, wp_ref, wg_ref, y_ref, lg_ref):
    x = x_ref[0]
    y = jnp.dot(_silu(x), wb_ref[...], preferred_element_type=_F32)
    y += jnp.dot(_basis_cat(x), wp_ref[...], preferred_element_type=_F32)
    y = _inorm_silu(y)
    y_ref[0] = y
    pooled = jnp.mean(y, axis=0, keepdims=True)
    lg_ref[0] = jnp.dot(pooled, wg_ref[...], preferred_element_type=_F32)


def _k1_call(x, wb, wp, wg):
    b, n, c = x.shape
    m = wb.shape[1]
    return pl.pallas_call(
        _k1_body,
        grid=(b,),
        in_specs=[
            pl.BlockSpec((1, n, c), lambda i: (i, 0, 0)),
            pl.BlockSpec((c, m), lambda i: (0, 0)),
            pl.BlockSpec((4 * c, m), lambda i: (0, 0)),
            pl.BlockSpec((m, 8), lambda i: (0, 0)),
        ],
        out_specs=[
            pl.BlockSpec((1, n, m), lambda i: (i, 0, 0)),
            pl.BlockSpec((1, 1, 8), lambda i: (i, 0, 0)),
        ],
        out_shape=[
            jax.ShapeDtypeStruct((b, n, m), _F32),
            jax.ShapeDtypeStruct((b, 1, 8), _F32),
        ],
    )(x, wb, wp, wg)


def _k3_body(x_ref, wb_ref, wp_ref, xid_ref, wd_ref, y_ref, pool_ref):
    x = x_ref[0]
    y = jnp.dot(_silu(x), wb_ref[...], preferred_element_type=_F32)
    y += jnp.dot(_basis_cat(x), wp_ref[...], preferred_element_type=_F32)
    y = _inorm_silu(y)
    y += jnp.dot(xid_ref[0], wd_ref[...], preferred_element_type=_F32)
    y_ref[0] = y
    pool_ref[0] = jnp.mean(y, axis=0, keepdims=True)


def _k3_call(x, wb, wp, xid, wd):
    b, n, c = x.shape
    m = wb.shape[1]
    cin = wd.shape[0]
    return pl.pallas_call(
        _k3_body,
        grid=(b,),
        in_specs=[
            pl.BlockSpec((1, n, c), lambda i: (i, 0, 0)),
            pl.BlockSpec((c, m), lambda i: (0, 0)),
            pl.BlockSpec((4 * c, m), lambda i: (0, 0)),
            pl.BlockSpec((1, n, cin), lambda i: (i, 0, 0)),
            pl.BlockSpec((cin, m), lambda i: (0, 0)),
        ],
        out_specs=[
            pl.BlockSpec((1, n, m), lambda i: (i, 0, 0)),
            pl.BlockSpec((1, 1, m), lambda i: (i, 0, 0)),
        ],
        out_shape=[
            jax.ShapeDtypeStruct((b, n, m), _F32),
            jax.ShapeDtypeStruct((b, 1, m), _F32),
        ],
    )(x, wb, wp, xid, wd)


def _k2_body(t_ref, mask_ref, g_ref, wc_ref, out_ref, acc_ref, *, m):
    tv = t_ref[0]
    for tap in range(9):
        xt = tv[:, tap * m:(tap + 1) * m]
        feat = jnp.concatenate(
            [_silu(xt), _basis_cat(xt) * mask_ref[:, tap:tap + 1]], axis=-1)
        c = jnp.dot(feat, wc_ref[tap], preferred_element_type=_F32)
        if tap == 0:
            acc_ref[...] = c
        else:
            acc_ref[...] += c
    yv = acc_ref[...]
    y = g_ref[0, 0, 0] * yv[:, 0:m]
    for e in range(1, 8):
        y += g_ref[0, 0, e] * yv[:, e * m:(e + 1) * m]
    out_ref[0] = _inorm_silu(y)


def _k2_call(taps, mask, gates, wc):
    b, n, m9 = taps.shape
    m = m9 // 9
    em = wc.shape[2]
    return pl.pallas_call(
        functools.partial(_k2_body, m=m),
        grid=(b,),
        in_specs=[
            pl.BlockSpec((1, n, m9), lambda i: (i, 0, 0)),
            pl.BlockSpec((n, 9), lambda i: (0, 0)),
            pl.BlockSpec((1, 1, 8), lambda i: (i, 0, 0)),
            pl.BlockSpec((9, 5 * m, em), lambda i: (0, 0, 0)),
        ],
        out_specs=pl.BlockSpec((1, n, m), lambda i: (i, 0, 0)),
        out_shape=jax.ShapeDtypeStruct((b, n, m), _F32),
        scratch_shapes=[pltpu.VMEM((n, em), _F32)],
    )(taps, mask, gates, wc)


def _router_sc(logits16):
    mesh = plsc.VectorSubcoreMesh(core_axis_name="c", subcore_axis_name="s")

    dnums = lax.GatherDimensionNumbers(
        offset_dims=(), collapsed_slice_dims=(0,), start_index_map=(0,))

    def _shuf(v, idx):
        return lax.gather(v, idx[:, None], dimension_numbers=dnums,
                          slice_sizes=(1,),
                          mode=lax.GatherScatterMode.PROMISE_IN_BOUNDS)

    def _bfly(v, op, io):
        for s in (8, 4, 2, 1):
            v = op(v, _shuf(v, io ^ s))
        return v

    @functools.partial(
        pl.kernel,
        mesh=mesh,
        out_type=(
            jax.ShapeDtypeStruct((8, 16), _F32),
            jax.ShapeDtypeStruct((16,), _F32),
        ),
        scratch_types=[
            pltpu.VMEM((8, 16), _F32),
            pltpu.VMEM((8, 16), _F32),
            pltpu.VMEM((16,), _F32),
        ],
    )
    def router(lg_hbm, gates_hbm, loss_hbm, lg_v, gt_v, ls_v):
        cid = lax.axis_index("c")
        sid = lax.axis_index("s")

        @pl.when(jnp.logical_and(cid == 0, sid == 0))
        def _():
            pltpu.sync_copy(lg_hbm, lg_v)
            io = lax.iota(jnp.int32, 16)
            imp = jnp.zeros((16,), _F32)
            for b in range(8):
                v = lg_v[b]
                m1 = _bfly(v, jnp.maximum, io)
                i1 = _bfly(jnp.where(v == m1, io, 16), jnp.minimum, io)
                sel1 = io == i1
                v2 = jnp.where(sel1, _NEG, v)
                m2 = _bfly(v2, jnp.maximum, io)
                i2 = _bfly(jnp.where(v2 == m2, io, 16), jnp.minimum, io)
                sel2 = io == i2
                z = jnp.exp(m2 - m1)
                g1 = 1.0 / (1.0 + z)
                row = jnp.where(sel1, g1, 0.0) + jnp.where(sel2, g1 * z, 0.0)
                gt_v[b] = row
                imp = imp + row
            mean = _bfly(imp, jnp.add, io) * 0.125
            d = jnp.where(io < 8, imp - mean, 0.0)
            var = _bfly(d * d, jnp.add, io) * 0.125
            ls_v[...] = var / (mean * mean + 1e-10)
            pltpu.sync_copy(gt_v, gates_hbm)
            pltpu.sync_copy(ls_v, loss_hbm)

    return router(logits16)


def _fc_body(p_ref, w_ref, b_ref, l_ref, out_ref, tot_ref):
    out_ref[...] = (jnp.dot(p_ref[...], w_ref[...],
                            preferred_element_type=_F32) + b_ref[...])
    tot_ref[...] = jnp.sum(l_ref[...]).reshape(1, 1)


def _fc_call(pooled, w, bvec, losses):
    return pl.pallas_call(
        _fc_body,
        out_shape=[
            jax.ShapeDtypeStruct((8, 1000), _F32),
            jax.ShapeDtypeStruct((1, 1), _F32),
        ],
    )(pooled, w, bvec, losses)


def _conv_taps(y, h, w, stride, neg=False):
    b, _, c = y.shape
    yr = y.reshape(b, h, w, c)
    pad = jnp.pad(yr, ((0, 0), (1, 1), (1, 1), (0, 0)),
                  constant_values=_NEG if neg else 0.0)
    oh, ow = (h + stride - 1) // stride, (w + stride - 1) // stride
    taps = []
    for ky in range(3):
        for kx in range(3):
            s = lax.slice(pad, (0, ky, kx, 0),
                          (b, ky + stride * (oh - 1) + 1,
                           kx + stride * (ow - 1) + 1, c),
                          (1, stride, stride, 1))
            taps.append(s.reshape(b, oh * ow, c))
    return jnp.concatenate(taps, axis=-1)


def _tap_mask(h, w, stride):
    oh, ow = (h + stride - 1) // stride, (w + stride - 1) // stride
    oy = jnp.arange(oh)
    ox = jnp.arange(ow)
    masks = []
    for ky in range(3):
        for kx in range(3):
            rv = ((stride * oy + ky - 1 >= 0) &
                  (stride * oy + ky - 1 <= h - 1)).astype(_F32)
            cv = ((stride * ox + kx - 1 >= 0) &
                  (stride * ox + kx - 1 <= w - 1)).astype(_F32)
            masks.append((rv[:, None] * cv[None, :]).reshape(oh * ow, 1))
    return jnp.concatenate(masks, axis=-1)


def _moe_weight(w2b, w2p):
    e, o, i, _, _ = w2b.shape
    wb = w2b.transpose(3, 4, 2, 0, 1).reshape(9, i, e * o)
    wp = w2p.transpose(3, 4, 2, 0, 1).reshape(9, 4 * i, e * o)
    return jnp.concatenate([wb, wp], axis=1)


def _block(x_in, p, stride, h):
    b, n, c = x_in.shape
    m = p['w1b'].shape[0]
    cout = p['w3b'].shape[0]
    y1, logits = _k1_call(
        x_in,
        p['w1b'][:, :, 0, 0].T,
        p['w1p'][:, :, 0, 0].T,
        p['wg'],
    )
    lg16 = jnp.pad(logits.reshape(b, 8), ((0, 0), (0, 8)),
                   constant_values=_NEG)
    gates16, lossv = _router_sc(lg16)
    gates = gates16[:, :8].reshape(b, 1, 8)
    taps = _conv_taps(y1, h, h, stride)
    mask = _tap_mask(h, h, stride)
    wc = _moe_weight(p['w2b'], p['w2p'])
    y2 = _k2_call(taps, mask, gates, wc)
    oh = (h + stride - 1) // stride
    xid = x_in.reshape(b, h, h, c)[:, ::stride, ::stride, :].reshape(
        b, oh * oh, c)
    y3, pooled = _k3_call(
        y2,
        p['w3b'][:, :, 0, 0].T,
        p['w3p'][:, :, 0, 0].T,
        xid,
        p['wd'][:, :, 0, 0].T,
    )
    return y3, lossv[0], pooled.reshape(b, cout), oh


def _stem_weights(stem_b, stem_p):
    wgm = jnp.zeros((16, 60, 32), _F32)
    for ky in range(7):
        dy, py = ky // 2, ky % 2
        for kx in range(7):
            dx, px = kx // 2, kx % 2
            g = dy * 4 + dx
            blk = (py * 2 + px) * 15
            wgm = wgm.at[g, blk:blk + 3, 0:16].set(stem_b[:, :, ky, kx].T)
            wgm = wgm.at[g, blk + 3:blk + 15, 16:32].set(
                stem_p[:, :, ky, kx].T)
    return wgm


def _stem_inputs(x):
    xpad = jnp.pad(x, ((0, 0), (0, 0), (3, 3), (3, 3)))
    phs = [xpad[:, :, py::2, px::2] for py in (0, 1) for px in (0, 1)]
    ph = jnp.stack(phs, axis=1)
    ph = ph.transpose(0, 3, 4, 1, 2).reshape(8, 115, 115, 12)
    ph = jnp.pad(ph, ((0, 0), (0, 1), (0, 0), (0, 0)))
    bands = [lax.slice(ph, (0, k * 16, 0, 0), (8, k * 16 + 20, 115, 12))
             for k in range(7)]
    return jnp.stack(bands, axis=1)


def kernel(x, params):
    ph_bands = _stem_inputs(x)
    wgm = _stem_weights(params['stem_b'], params['stem_p'])
    y0 = _stem_call(ph_bands, wgm)
    ptaps = _conv_taps(y0, 112, 112, 2, neg=True)
    out = _pool_call(ptaps, y0)
    h = 56
    losses = []
    pooled = None
    for p, s in zip(params['blocks'], (1, 2, 2, 2)):
        out, lv, pooled, h = _block(out, p, s, h)
        losses.append(lv)
    logits, tot = _fc_call(
        pooled,
        params['fc_w'],
        params['fc_b'].reshape(1, 1000),
        jnp.stack(losses).reshape(1, 4),
    )
    return logits, tot.reshape(())
